# baseline (speedup 1.0000x reference)
import functools

import jax
import jax.numpy as jnp
from jax import lax
from jax.experimental import pallas as pl
from jax.experimental.pallas import tpu as pltpu
from jax.experimental.pallas import tpu_sc as plsc

VOCAB = 100000
EMBED = 128
B = 16384
NNEG = 20
NCTX = NNEG + 1
NLANE = 16
NREG = EMBED // NLANE

NC = 2
NS = 16
NW = NC * NS
RW = B // NW
C = 16
NCHUNK = RW // C
CI = C * NCTX

_DNUMS = lax.GatherDimensionNumbers(
    offset_dims=(), collapsed_slice_dims=(0,), start_index_map=(0,))


def _lane_rot(p, sh):
  perm = ((lax.iota(jnp.int32, NLANE) + sh) % NLANE)[:, None]
  return lax.gather(p, perm, _DNUMS, (1,),
                    mode=lax.GatherScatterMode.PROMISE_IN_BOUNDS)


def _allsum(p):
  for sh in (8, 4, 2, 1):
    p = p + _lane_rot(p, sh)
  return p


def _logsig(z):
  u = jnp.exp(-jnp.abs(z))
  y = 1.0 + u
  w = u * (1.0 - u * (0.5 - u * (1.0 / 3.0)))
  w = w + y * jnp.exp(-w) - 1.0
  w = w + y * jnp.exp(-w) - 1.0
  w = w + y * jnp.exp(-w) - 1.0
  return jnp.minimum(z, 0.0) - w


def _sc_loss(word_pos, cidx, word_table, ctx_table):
  mesh = plsc.VectorSubcoreMesh(core_axis_name="c", subcore_axis_name="s")

  @functools.partial(
      pl.kernel,
      mesh=mesh,
      out_type=jax.ShapeDtypeStruct((B,), jnp.float32),
      scratch_types=[
          pltpu.VMEM((RW,), jnp.int32),
          pltpu.VMEM((RW * NCTX,), jnp.int32),
          pltpu.VMEM((C, EMBED), jnp.float32),
          pltpu.VMEM((C, EMBED), jnp.float32),
          pltpu.VMEM((CI, EMBED), jnp.float32),
          pltpu.VMEM((CI, EMBED), jnp.float32),
          pltpu.VMEM((RW,), jnp.float32),
          pltpu.SemaphoreType.DMA,
          pltpu.SemaphoreType.DMA,
      ],
  )
  def k(wp_hbm, cidx_hbm, wt_hbm, ct_hbm, out_hbm,
        widx_v, cidx_v, wr0, wr1, cr0, cr1, lbuf_v, sem0, sem1):
    wid = lax.axis_index("s") * NC + lax.axis_index("c")
    base = pl.multiple_of(wid * RW, RW)
    pltpu.sync_copy(wp_hbm.at[pl.ds(base, RW)], widx_v)
    pltpu.sync_copy(cidx_hbm.at[pl.ds(base * NCTX, RW * NCTX)], cidx_v)

    lane = lax.iota(jnp.int32, NLANE)

    def descr(c, wr, cr, sem):
      cb = pl.multiple_of(c * C, C)
      cib = pl.multiple_of(c * CI, CI)
      return (
          (wt_hbm.at[widx_v.at[pl.ds(cb, C)]], wr, sem),
          (ct_hbm.at[cidx_v.at[pl.ds(cib, 128)]], cr.at[pl.ds(0, 128)], sem),
          (ct_hbm.at[cidx_v.at[pl.ds(cib + 128, 128)]],
           cr.at[pl.ds(128, 128)], sem),
          (ct_hbm.at[cidx_v.at[pl.ds(cib + 256, CI - 256)]],
           cr.at[pl.ds(256, CI - 256)], sem),
      )

    def issue(c, wr, cr, sem):
      for d in descr(c, wr, cr, sem):
        pltpu.async_copy(*d)

    def wait(c, wr, cr, sem):
      for d in descr(c, wr, cr, sem):
        pltpu.make_async_copy(*d).wait()

    def compute(c, wr, cr):
      cb = pl.multiple_of(c * C, C)

      def row_body(i, acc):
        w = [wr[i, pl.ds(r * NLANE, NLANE)] for r in range(NREG)]

        def dot(crow):
          p = w[0] * crow[pl.ds(0, NLANE)]
          for r in range(1, NREG):
            p = p + w[r] * crow[pl.ds(r * NLANE, NLANE)]
          return _allsum(p)

        L = _logsig(dot(cr.at[i * NCTX]))
        for j in range(1, NCTX):
          L = L + _logsig(-dot(cr.at[i * NCTX + j]))
        return jnp.where(lane == i, -L, acc)

      lbuf_v[pl.ds(cb, NLANE)] = lax.fori_loop(
          0, C, row_body, jnp.zeros((NLANE,), jnp.float32))

    issue(0, wr0, cr0, sem0)

    def chunk_body(c, _):
      @pl.when(c % 2 == 0)
      def _():
        @pl.when(c + 1 < NCHUNK)
        def _():
          issue(c + 1, wr1, cr1, sem1)
        wait(c, wr0, cr0, sem0)
        compute(c, wr0, cr0)

      @pl.when(c % 2 == 1)
      def _():
        @pl.when(c + 1 < NCHUNK)
        def _():
          issue(c + 1, wr0, cr0, sem0)
        wait(c, wr1, cr1, sem1)
        compute(c, wr1, cr1)

      return 0

    lax.fori_loop(0, NCHUNK, chunk_body, 0)
    pltpu.sync_copy(lbuf_v, out_hbm.at[pl.ds(base, RW)])

  return k(word_pos, cidx, word_table, ctx_table)


def kernel(word_pos, ctx_pos, neg_ctx_pos, word_table, ctx_table):
  word_pos = word_pos.astype(jnp.int32)
  cidx = jnp.concatenate(
      [ctx_pos.astype(jnp.int32)[:, None], neg_ctx_pos.astype(jnp.int32)],
      axis=1).reshape(-1)
  return _sc_loss(word_pos, cidx, word_table, ctx_table)


# speedup vs baseline: 3.3834x; 3.3834x over previous
import functools

import jax
import jax.numpy as jnp
from jax import lax
from jax.experimental import pallas as pl
from jax.experimental.pallas import tpu as pltpu
from jax.experimental.pallas import tpu_sc as plsc

VOCAB = 100000
EMBED = 128
B = 16384
NNEG = 20
NCTX = NNEG + 1
NLANE = 16
NREG = EMBED // NLANE
SROW = 32

NC = 2
NS = 16
NW = NC * NS
RW = B // NW
C = 16
NCHUNK = RW // C
CI = C * NCTX

_DNUMS = lax.GatherDimensionNumbers(
    offset_dims=(), collapsed_slice_dims=(0,), start_index_map=(0,))


def _lane_rot(p, sh):
  perm = ((lax.iota(jnp.int32, NLANE) + sh) % NLANE)[:, None]
  return lax.gather(p, perm, _DNUMS, (1,),
                    mode=lax.GatherScatterMode.PROMISE_IN_BOUNDS)


def _allsum(p):
  for sh in (8, 4, 2, 1):
    p = p + _lane_rot(p, sh)
  return p


def _sc_scores(word_pos, cidx, word_table, ctx_table):
  mesh = plsc.VectorSubcoreMesh(core_axis_name="c", subcore_axis_name="s")

  @functools.partial(
      pl.kernel,
      mesh=mesh,
      out_type=jax.ShapeDtypeStruct((B * SROW,), jnp.float32),
      scratch_types=[
          pltpu.VMEM((RW,), jnp.int32),
          pltpu.VMEM((RW * NCTX,), jnp.int32),
          pltpu.VMEM((C, EMBED), jnp.float32),
          pltpu.VMEM((C, EMBED), jnp.float32),
          pltpu.VMEM((CI, EMBED), jnp.float32),
          pltpu.VMEM((CI, EMBED), jnp.float32),
          pltpu.VMEM((RW * SROW,), jnp.float32),
          pltpu.SemaphoreType.DMA,
          pltpu.SemaphoreType.DMA,
      ],
  )
  def k(wp_hbm, cidx_hbm, wt_hbm, ct_hbm, out_hbm,
        widx_v, cidx_v, wr0, wr1, cr0, cr1, sbuf_v, sem0, sem1):
    wid = lax.axis_index("s") * NC + lax.axis_index("c")
    base = pl.multiple_of(wid * RW, RW)
    pltpu.sync_copy(wp_hbm.at[pl.ds(base, RW)], widx_v)
    pltpu.sync_copy(cidx_hbm.at[pl.ds(base * NCTX, RW * NCTX)], cidx_v)

    lane = lax.iota(jnp.int32, NLANE)

    def descr(c, wr, cr, sem):
      cb = pl.multiple_of(c * C, C)
      cib = pl.multiple_of(c * CI, CI)
      return (
          (wt_hbm.at[widx_v.at[pl.ds(cb, C)]], wr, sem),
          (ct_hbm.at[cidx_v.at[pl.ds(cib, 128)]], cr.at[pl.ds(0, 128)], sem),
          (ct_hbm.at[cidx_v.at[pl.ds(cib + 128, 128)]],
           cr.at[pl.ds(128, 128)], sem),
          (ct_hbm.at[cidx_v.at[pl.ds(cib + 256, CI - 256)]],
           cr.at[pl.ds(256, CI - 256)], sem),
      )

    def issue(c, wr, cr, sem):
      for d in descr(c, wr, cr, sem):
        pltpu.async_copy(*d)

    def wait(c, wr, cr, sem):
      for d in descr(c, wr, cr, sem):
        pltpu.make_async_copy(*d).wait()

    def compute(c, wr, cr):
      cb = pl.multiple_of(c * C, C)

      def row_body(i, _):
        w = [wr[i, pl.ds(r * NLANE, NLANE)] for r in range(NREG)]
        s_lo = jnp.zeros((NLANE,), jnp.float32)
        s_hi = jnp.zeros((NLANE,), jnp.float32)
        for j in range(NCTX):
          crow = cr.at[i * NCTX + j]
          p = w[0] * crow[pl.ds(0, NLANE)]
          for r in range(1, NREG):
            p = p + w[r] * crow[pl.ds(r * NLANE, NLANE)]
          tot = _allsum(p)
          if j < NLANE:
            s_lo = jnp.where(lane == j, tot, s_lo)
          else:
            s_hi = jnp.where(lane == (j - NLANE), tot, s_hi)
        sb = (cb + i) * SROW
        sbuf_v[pl.ds(sb, NLANE)] = s_lo
        sbuf_v[pl.ds(sb + NLANE, NLANE)] = s_hi
        return 0

      lax.fori_loop(0, C, row_body, 0)

    issue(0, wr0, cr0, sem0)

    def chunk_body(c, _):
      @pl.when(c % 2 == 0)
      def _():
        @pl.when(c + 1 < NCHUNK)
        def _():
          issue(c + 1, wr1, cr1, sem1)
        wait(c, wr0, cr0, sem0)
        compute(c, wr0, cr0)

      @pl.when(c % 2 == 1)
      def _():
        @pl.when(c + 1 < NCHUNK)
        def _():
          issue(c + 1, wr0, cr0, sem0)
        wait(c, wr1, cr1, sem1)
        compute(c, wr1, cr1)

      return 0

    lax.fori_loop(0, NCHUNK, chunk_body, 0)
    pltpu.sync_copy(sbuf_v, out_hbm.at[pl.ds(base * SROW, RW * SROW)])

  return k(word_pos, cidx, word_table, ctx_table)


def _tc_finish(scores):
  def body(s_ref, o_ref):
    s = s_ref[...]
    pos = s[:, 0:1]
    neg = -s[:, 1:NCTX]

    def logsig(x):
      return jnp.minimum(x, 0.0) - jnp.log1p(jnp.exp(-jnp.abs(x)))

    o_ref[...] = -(logsig(pos)[:, 0] + jnp.sum(logsig(neg), axis=1))

  return pl.pallas_call(
      body,
      out_shape=jax.ShapeDtypeStruct((B,), jnp.float32),
  )(scores)


def kernel(word_pos, ctx_pos, neg_ctx_pos, word_table, ctx_table):
  word_pos = word_pos.astype(jnp.int32)
  cidx = jnp.concatenate(
      [ctx_pos.astype(jnp.int32)[:, None], neg_ctx_pos.astype(jnp.int32)],
      axis=1).reshape(-1)
  scores = _sc_scores(word_pos, cidx, word_table, ctx_table)
  return _tc_finish(scores.reshape(B, SROW))


# speedup vs baseline: 3.8559x; 1.1397x over previous
import functools

import jax
import jax.numpy as jnp
from jax import lax
from jax.experimental import pallas as pl
from jax.experimental.pallas import tpu as pltpu
from jax.experimental.pallas import tpu_sc as plsc

VOCAB = 100000
EMBED = 128
B = 16384
NNEG = 20
NCTX = NNEG + 1
NLANE = 16
NREG = EMBED // NLANE
SROW = 32

NC = 2
NS = 16
NW = NC * NS
RW = B // NW
C = 16
NCHUNK = RW // C
NI = C * NNEG

_DNUMS = lax.GatherDimensionNumbers(
    offset_dims=(), collapsed_slice_dims=(0,), start_index_map=(0,))


def _lane_rot(p, sh):
  perm = ((lax.iota(jnp.int32, NLANE) + sh) % NLANE)[:, None]
  return lax.gather(p, perm, _DNUMS, (1,),
                    mode=lax.GatherScatterMode.PROMISE_IN_BOUNDS)


def _allsum(p):
  for sh in (8, 4, 2, 1):
    p = p + _lane_rot(p, sh)
  return p


def _sc_scores(word_pos, ctx_pos, neg_flat, word_table, ctx_table):
  mesh = plsc.VectorSubcoreMesh(core_axis_name="c", subcore_axis_name="s")

  @functools.partial(
      pl.kernel,
      mesh=mesh,
      out_type=jax.ShapeDtypeStruct((B * SROW,), jnp.float32),
      scratch_types=[
          pltpu.VMEM((RW,), jnp.int32),
          pltpu.VMEM((RW,), jnp.int32),
          pltpu.VMEM((RW * NNEG,), jnp.int32),
          pltpu.VMEM((C, EMBED), jnp.float32),
          pltpu.VMEM((C, EMBED), jnp.float32),
          pltpu.VMEM((C, EMBED), jnp.float32),
          pltpu.VMEM((C, EMBED), jnp.float32),
          pltpu.VMEM((NI, EMBED), jnp.float32),
          pltpu.VMEM((NI, EMBED), jnp.float32),
          pltpu.VMEM((RW * SROW,), jnp.float32),
          pltpu.SemaphoreType.DMA,
          pltpu.SemaphoreType.DMA,
      ],
  )
  def k(wp_hbm, cp_hbm, np_hbm, wt_hbm, ct_hbm, out_hbm,
        widx_v, pidx_v, nidx_v, wr0, wr1, pr0, pr1, nr0, nr1,
        sbuf_v, sem0, sem1):
    wid = lax.axis_index("s") * NC + lax.axis_index("c")
    base = pl.multiple_of(wid * RW, RW)
    pltpu.sync_copy(wp_hbm.at[pl.ds(base, RW)], widx_v)
    pltpu.sync_copy(cp_hbm.at[pl.ds(base, RW)], pidx_v)
    pltpu.sync_copy(np_hbm.at[pl.ds(base * NNEG, RW * NNEG)], nidx_v)

    lane = lax.iota(jnp.int32, NLANE)

    def descr(c, wr, pr, nr, sem):
      cb = pl.multiple_of(c * C, C)
      nb = pl.multiple_of(c * NI, NI)
      return (
          (wt_hbm.at[widx_v.at[pl.ds(cb, C)]], wr, sem),
          (ct_hbm.at[pidx_v.at[pl.ds(cb, C)]], pr, sem),
          (ct_hbm.at[nidx_v.at[pl.ds(nb, 128)]], nr.at[pl.ds(0, 128)], sem),
          (ct_hbm.at[nidx_v.at[pl.ds(nb + 128, 128)]],
           nr.at[pl.ds(128, 128)], sem),
          (ct_hbm.at[nidx_v.at[pl.ds(nb + 256, NI - 256)]],
           nr.at[pl.ds(256, NI - 256)], sem),
      )

    def issue(c, wr, pr, nr, sem):
      for d in descr(c, wr, pr, nr, sem):
        pltpu.async_copy(*d)

    def wait(c, wr, pr, nr, sem):
      for d in descr(c, wr, pr, nr, sem):
        pltpu.make_async_copy(*d).wait()

    def compute(c, wr, pr, nr):
      cb = pl.multiple_of(c * C, C)

      def row_body(i, _):
        w = [wr[i, pl.ds(r * NLANE, NLANE)] for r in range(NREG)]

        def dot(crow):
          p = w[0] * crow[pl.ds(0, NLANE)]
          for r in range(1, NREG):
            p = p + w[r] * crow[pl.ds(r * NLANE, NLANE)]
          return _allsum(p)

        s_lo = jnp.where(lane == 0, dot(pr.at[i]), 0.0)
        s_hi = jnp.zeros((NLANE,), jnp.float32)
        for j in range(1, NCTX):
          tot = dot(nr.at[i * NNEG + (j - 1)])
          if j < NLANE:
            s_lo = jnp.where(lane == j, tot, s_lo)
          else:
            s_hi = jnp.where(lane == (j - NLANE), tot, s_hi)
        sb = (cb + i) * SROW
        sbuf_v[pl.ds(sb, NLANE)] = s_lo
        sbuf_v[pl.ds(sb + NLANE, NLANE)] = s_hi
        return 0

      lax.fori_loop(0, C, row_body, 0)

    issue(0, wr0, pr0, nr0, sem0)

    def chunk_body(c, _):
      @pl.when(c % 2 == 0)
      def _():
        @pl.when(c + 1 < NCHUNK)
        def _():
          issue(c + 1, wr1, pr1, nr1, sem1)
        wait(c, wr0, pr0, nr0, sem0)
        compute(c, wr0, pr0, nr0)

      @pl.when(c % 2 == 1)
      def _():
        @pl.when(c + 1 < NCHUNK)
        def _():
          issue(c + 1, wr0, pr0, nr0, sem0)
        wait(c, wr1, pr1, nr1, sem1)
        compute(c, wr1, pr1, nr1)

      return 0

    lax.fori_loop(0, NCHUNK, chunk_body, 0)
    pltpu.sync_copy(sbuf_v, out_hbm.at[pl.ds(base * SROW, RW * SROW)])

  return k(word_pos, ctx_pos, neg_flat, word_table, ctx_table)


TCR = B * SROW // 128
GRP = 128 // SROW


def _tc_finish(scores128):
  def body(s_ref, o_ref):
    x = s_ref[...]
    col = lax.broadcasted_iota(jnp.int32, (TCR, 128), 1)
    m = col % SROW
    val = jnp.where(m == 0, x, -x)
    ls = jnp.minimum(val, 0.0) - jnp.log1p(jnp.exp(-jnp.abs(val)))
    contrib = jnp.where(m <= NNEG, ls, 0.0)
    gi = lax.broadcasted_iota(jnp.int32, (128, GRP), 0) // SROW
    gj = lax.broadcasted_iota(jnp.int32, (128, GRP), 1)
    sel = jnp.where(gi == gj, -1.0, 0.0).astype(jnp.float32)
    o_ref[...] = jnp.dot(contrib, sel, preferred_element_type=jnp.float32)

  return pl.pallas_call(
      body,
      out_shape=jax.ShapeDtypeStruct((TCR, GRP), jnp.float32),
  )(scores128)


def kernel(word_pos, ctx_pos, neg_ctx_pos, word_table, ctx_table):
  word_pos = word_pos.astype(jnp.int32)
  ctx_pos = ctx_pos.astype(jnp.int32)
  neg_flat = neg_ctx_pos.astype(jnp.int32).reshape(-1)
  scores = _sc_scores(word_pos, ctx_pos, neg_flat, word_table, ctx_table)
  return _tc_finish(scores.reshape(TCR, 128)).reshape(B)
